# Initial kernel scaffold; baseline (speedup 1.0000x reference)
import jax
import jax.numpy as jnp
from jax.experimental import pallas as pl


def kernel(x, edge_index, edge_attr, W_edge, b_edge, W1, b1, W2, b2, gamma, beta):
    raise NotImplementedError("write your pallas kernel here")


# speedup vs baseline: 2.6751x; 2.6751x over previous
import functools

import jax
import jax.numpy as jnp
from jax import lax
from jax.experimental import pallas as pl
from jax.experimental.pallas import tpu as pltpu
from jax.experimental.pallas import tpu_sc as plsc


def _edge_linear_body(ea_ref, w_ref, b_ref, out_ref):
    out_ref[...] = (
        jnp.dot(ea_ref[...], w_ref[...], preferred_element_type=jnp.float32)
        + b_ref[...]
    )


def _edge_linear(ea_p, w_p, b):
    E, EDP = ea_p.shape
    D = w_p.shape[1]
    BE = 3200
    assert E % BE == 0
    return pl.pallas_call(
        _edge_linear_body,
        grid=(E // BE,),
        in_specs=[
            pl.BlockSpec((BE, EDP), lambda i: (i, 0)),
            pl.BlockSpec((EDP, D), lambda i: (0, 0)),
            pl.BlockSpec((1, D), lambda i: (0, 0)),
        ],
        out_specs=pl.BlockSpec((BE, D), lambda i: (i, 0)),
        out_shape=jax.ShapeDtypeStruct((E, D), jnp.float32),
    )(ea_p, w_p, b)


def _sc_aggregate(x, e, src, dst):
    N, D = x.shape
    E = e.shape[0]
    info = plsc.get_sparse_core_info()
    NC, NS, L = info.num_cores, info.num_subcores, info.num_lanes
    NW = NC * NS
    C = 128
    assert E % C == 0
    n_chunks = E // C
    niter = (n_chunks + NW - 1) // NW
    RC = 128
    n_row_chunks = N // RC
    row_tail = N - n_row_chunks * RC
    assert row_tail % 8 == 0
    nzi = (n_row_chunks + NS - 1) // NS

    mesh = plsc.VectorSubcoreMesh(core_axis_name="c", subcore_axis_name="s")

    @functools.partial(
        pl.kernel,
        mesh=mesh,
        out_type=jax.ShapeDtypeStruct((NC, N, D), jnp.float32),
        scratch_types=[
            pltpu.VMEM((C,), jnp.int32),
            pltpu.VMEM((C,), jnp.int32),
            pltpu.VMEM((C, D), jnp.float32),
            pltpu.VMEM((C, D), jnp.float32),
            pltpu.VMEM_SHARED((N, D), jnp.float32),
            pltpu.SemaphoreType.DMA,
        ],
    )
    def k(x_hbm, e_hbm, src_hbm, dst_hbm, out_hbm, src_v, dst_v, e_v, m_v,
          aggr_sh, sem):
        cid = lax.axis_index("c")
        sid = lax.axis_index("s")
        wid = sid * NC + cid

        def zrow(r, carry):
            for dsub in range(D // L):
                m_v[r, pl.ds(dsub * L, L)] = jnp.zeros((L,), jnp.float32)
            return carry

        lax.fori_loop(0, RC, zrow, 0)

        def zchunk(i, carry):
            c = i * NS + sid

            @pl.when(c < n_row_chunks)
            def _():
                pltpu.sync_copy(m_v, aggr_sh.at[pl.ds(c * RC, RC)])

            return carry

        lax.fori_loop(0, nzi, zchunk, 0)
        if row_tail:
            @pl.when(sid == 0)
            def _():
                pltpu.sync_copy(m_v.at[pl.ds(0, row_tail)],
                                aggr_sh.at[pl.ds(n_row_chunks * RC, row_tail)])
        plsc.subcore_barrier()

        def body(g, carry):
            chunk = g * NW + wid

            @pl.when(chunk < n_chunks)
            def _():
                ebase = chunk * C
                pltpu.sync_copy(src_hbm.at[pl.ds(ebase, C)], src_v)
                pltpu.sync_copy(dst_hbm.at[pl.ds(ebase, C)], dst_v)
                pltpu.sync_copy(e_hbm.at[pl.ds(ebase, C)], e_v)
                pltpu.async_copy(x_hbm.at[src_v], m_v, sem).wait()

                def crow(r, c2):
                    for dsub in range(D // L):
                        sl = pl.ds(dsub * L, L)
                        m_v[r, sl] = jnp.maximum(m_v[r, sl] + e_v[r, sl], 0.0)
                    return c2

                lax.fori_loop(0, C, crow, 0)
                pltpu.sync_copy(m_v, aggr_sh.at[dst_v], add=True)

            return carry

        lax.fori_loop(0, niter, body, 0)
        plsc.subcore_barrier()

        def dchunk(i, carry):
            c = i * NS + sid

            @pl.when(c < n_row_chunks)
            def _():
                r0 = c * RC
                pltpu.sync_copy(aggr_sh.at[pl.ds(r0, RC)], m_v)
                pltpu.sync_copy(m_v, out_hbm.at[cid, pl.ds(r0, RC)])

            return carry

        lax.fori_loop(0, nzi, dchunk, 0)
        if row_tail:
            @pl.when(sid == 0)
            def _():
                r0 = n_row_chunks * RC
                pltpu.sync_copy(aggr_sh.at[pl.ds(r0, row_tail)],
                                m_v.at[pl.ds(0, row_tail)])
                pltpu.sync_copy(m_v.at[pl.ds(0, row_tail)],
                                out_hbm.at[cid, pl.ds(r0, row_tail)])

    return k(x, e, src, dst)


def _mlp_stats_body(x_ref, a0_ref, a1_ref, w1_ref, b1_ref, w2_ref, b2_ref,
                    h_ref, s_ref, sq_ref):
    i = pl.program_id(0)
    out = x_ref[...] + a0_ref[...] + a1_ref[...]
    h1 = jnp.maximum(
        jnp.dot(out, w1_ref[...], preferred_element_type=jnp.float32)
        + b1_ref[...], 0.0)
    h2 = jnp.maximum(
        jnp.dot(h1, w2_ref[...], preferred_element_type=jnp.float32)
        + b2_ref[...], 0.0)
    h_ref[...] = h2
    s = jnp.sum(h2, axis=0, keepdims=True)
    sq = jnp.sum(h2 * h2, axis=0, keepdims=True)

    @pl.when(i == 0)
    def _():
        s_ref[...] = s
        sq_ref[...] = sq

    @pl.when(i != 0)
    def _():
        s_ref[...] += s
        sq_ref[...] += sq


def _mlp_stats(x, a0, a1, w1, b1, w2, b2):
    N, D = x.shape
    H = w1.shape[1]
    BN = 2000
    assert N % BN == 0
    row = lambda i: (i, 0)
    fixed = lambda i: (0, 0)
    return pl.pallas_call(
        _mlp_stats_body,
        grid=(N // BN,),
        in_specs=[
            pl.BlockSpec((BN, D), row),
            pl.BlockSpec((BN, D), row),
            pl.BlockSpec((BN, D), row),
            pl.BlockSpec((D, H), fixed),
            pl.BlockSpec((1, H), fixed),
            pl.BlockSpec((H, H), fixed),
            pl.BlockSpec((1, H), fixed),
        ],
        out_specs=[
            pl.BlockSpec((BN, H), row),
            pl.BlockSpec((1, H), fixed),
            pl.BlockSpec((1, H), fixed),
        ],
        out_shape=[
            jax.ShapeDtypeStruct((N, H), jnp.float32),
            jax.ShapeDtypeStruct((1, H), jnp.float32),
            jax.ShapeDtypeStruct((1, H), jnp.float32),
        ],
    )(x, a0, a1, w1, b1, w2, b2)


def _bn_body(n_total, h_ref, s_ref, sq_ref, g_ref, b_ref, out_ref):
    mean = s_ref[...] / n_total
    var = sq_ref[...] / n_total - mean * mean
    inv = lax.rsqrt(var + 1e-5)
    out_ref[...] = (h_ref[...] - mean) * (inv * g_ref[...]) + b_ref[...]


def _bn_apply(h, s, sq, gamma, beta):
    N, H = h.shape
    BN = 2000
    row = lambda i: (i, 0)
    fixed = lambda i: (0, 0)
    return pl.pallas_call(
        functools.partial(_bn_body, float(N)),
        grid=(N // BN,),
        in_specs=[
            pl.BlockSpec((BN, H), row),
            pl.BlockSpec((1, H), fixed),
            pl.BlockSpec((1, H), fixed),
            pl.BlockSpec((1, H), fixed),
            pl.BlockSpec((1, H), fixed),
        ],
        out_specs=pl.BlockSpec((BN, H), row),
        out_shape=jax.ShapeDtypeStruct((N, H), jnp.float32),
    )(h, s, sq, gamma, beta)


def kernel(x, edge_index, edge_attr, W_edge, b_edge, W1, b1, W2, b2, gamma,
           beta):
    N, D = x.shape
    E, ED = edge_attr.shape
    EDP = 16
    src = edge_index[0]
    dst = edge_index[1]
    ea_p = jnp.pad(edge_attr, ((0, 0), (0, EDP - ED)))
    w_p = jnp.pad(W_edge, ((0, EDP - ED), (0, 0)))
    e = _edge_linear(ea_p, w_p, b_edge.reshape(1, D))
    aggr = _sc_aggregate(x, e, src, dst)
    h, s, sq = _mlp_stats(x, aggr[0], aggr[1], W1, b1.reshape(1, -1), W2,
                          b2.reshape(1, -1))
    return _bn_apply(h, s, sq, gamma.reshape(1, -1), beta.reshape(1, -1))
